# Initial kernel scaffold; baseline (speedup 1.0000x reference)
import jax
import jax.numpy as jnp
from jax.experimental import pallas as pl


def kernel(nodes, coord, edges, edge_attr, eW1, eb1, eW2, eb2, aW1, ab1, aW2, ab2, cW1, cb1, cW2, cb2, nW1, nb1, nW2, nb2):
    raise NotImplementedError("write your pallas kernel here")


# speedup vs baseline: 3.3984x; 3.3984x over previous
import functools

import jax
import jax.numpy as jnp
from jax import lax
from jax.experimental import pallas as pl
from jax.experimental.pallas import tpu as pltpu
from jax.experimental.pallas import tpu_sc as plsc

N = 10000
E = 320000
D = 128
H = 128
DE = 16
CP = 16

NC, NS = 2, 16
NW = NC * NS
EPW = E // NW
CH = 128
NFULL = EPW // CH
TAIL = EPW - NFULL * CH
RPT = N // NW
RPS = N // NS

_f32 = jnp.float32
_i32 = jnp.int32


def _gather_body(nodes_h, coordp_h, row_h, col_h,
                 src_o, dst_o, srcc_o, dstc_o,
                 idxr, idxc, bs, bd, bsc, bdc,
                 tidxr, tidxc, tbs, tbd, tbsc, tbdc,
                 s1, s2, s3, s4):
    c = lax.axis_index("c")
    s = lax.axis_index("s")
    wid = s * NC + c
    base0 = wid * EPW

    def chunk(base, ir, ic, b1, b2, b3, b4, sz):
        pltpu.sync_copy(row_h.at[pl.ds(base, sz)], ir)
        pltpu.sync_copy(col_h.at[pl.ds(base, sz)], ic)
        d1 = pltpu.async_copy(nodes_h.at[ir], b1, s1)
        d2 = pltpu.async_copy(nodes_h.at[ic], b2, s2)
        d3 = pltpu.async_copy(coordp_h.at[ir], b3, s3)
        d4 = pltpu.async_copy(coordp_h.at[ic], b4, s4)
        d1.wait()
        d2.wait()
        d3.wait()
        d4.wait()
        pltpu.sync_copy(b1, src_o.at[pl.ds(base, sz)])
        pltpu.sync_copy(b2, dst_o.at[pl.ds(base, sz)])
        pltpu.sync_copy(b3, srcc_o.at[pl.ds(base, sz)])
        pltpu.sync_copy(b4, dstc_o.at[pl.ds(base, sz)])

    def body(i, _):
        chunk(base0 + i * CH, idxr, idxc, bs, bd, bsc, bdc, CH)
        return _

    lax.fori_loop(0, NFULL, body, None)
    if TAIL:
        chunk(base0 + NFULL * CH, tidxr, tidxc, tbs, tbd, tbsc, tbdc, TAIL)


def _sc_gather(nodes, coordp, row, col):
    fn = pl.kernel(
        _gather_body,
        out_type=[
            jax.ShapeDtypeStruct((E, D), _f32),
            jax.ShapeDtypeStruct((E, D), _f32),
            jax.ShapeDtypeStruct((E, CP), _f32),
            jax.ShapeDtypeStruct((E, CP), _f32),
        ],
        mesh=plsc.VectorSubcoreMesh(core_axis_name="c", subcore_axis_name="s",
                                    num_cores=NC, num_subcores=NS),
        scratch_types=[
            pltpu.VMEM((CH,), _i32), pltpu.VMEM((CH,), _i32),
            pltpu.VMEM((CH, D), _f32), pltpu.VMEM((CH, D), _f32),
            pltpu.VMEM((CH, CP), _f32), pltpu.VMEM((CH, CP), _f32),
            pltpu.VMEM((TAIL,), _i32), pltpu.VMEM((TAIL,), _i32),
            pltpu.VMEM((TAIL, D), _f32), pltpu.VMEM((TAIL, D), _f32),
            pltpu.VMEM((TAIL, CP), _f32), pltpu.VMEM((TAIL, CP), _f32),
            pltpu.SemaphoreType.DMA, pltpu.SemaphoreType.DMA,
            pltpu.SemaphoreType.DMA, pltpu.SemaphoreType.DMA,
        ],
        compiler_params=pltpu.CompilerParams(use_tc_tiling_on_sc=False),
    )
    return fn(nodes, coordp, row, col)


def _edge_mlp_body(src_r, dst_r, srcc_r, dstc_r, ea_r,
                   eW1s_r, eW1d_r, eW1r_r, eW1e_r, eb1_r, eW2_r, eb2_r,
                   aW1s_r, aW1d_r, aW1e_r, ab1_r, aW2_r, ab2_r,
                   cW1_r, cb1_r, cW2_r, cb2_r,
                   ef_o, tr_o):
    src = src_r[...]
    dst = dst_r[...]
    ea = ea_r[...]
    cd = srcc_r[...] - dstc_r[...]
    radial = jnp.sum(cd * cd, axis=1, keepdims=True)

    def mm(a, b):
        return jnp.dot(a, b, preferred_element_type=_f32)

    def tshrink(x):
        return x - jnp.tanh(x)

    h = (mm(src, eW1s_r[...]) + mm(dst, eW1d_r[...]) + radial * eW1r_r[...]
         + mm(ea, eW1e_r[...]) + eb1_r[...])
    h = tshrink(h)
    h = tshrink(mm(h, eW2_r[...]) + eb2_r[...])

    a = (mm(src, aW1s_r[...]) + mm(dst, aW1d_r[...]) + mm(ea, aW1e_r[...])
         + ab1_r[...])
    a = tshrink(a)
    a = mm(a, aW2_r[...]) + ab2_r[...]
    a = 1.0 / (1.0 + jnp.exp(-a))

    ef = h * a
    cc = tshrink(mm(ef, cW1_r[...]) + cb1_r[...])
    cc = mm(cc, cW2_r[...]) + cb2_r[...]

    ef_o[...] = ef
    tr_o[...] = cd * cc


def _edge_mlp(src, dst, srcc, dstc, edge_attr, w):
    BE = 1280
    grid = (E // BE,)

    def eb(i):
        return (i, 0)

    def wb(i):
        return (0, 0)

    ebs_d = pl.BlockSpec((BE, D), eb)
    ebs_c = pl.BlockSpec((BE, CP), eb)
    ebs_e = pl.BlockSpec((BE, DE), eb)

    def wspec(shape):
        return pl.BlockSpec(shape, wb)

    in_specs = [ebs_d, ebs_d, ebs_c, ebs_c, ebs_e,
                wspec((D, H)), wspec((D, H)), wspec((1, H)), wspec((DE, H)),
                wspec((1, H)), wspec((H, H)), wspec((1, H)),
                wspec((D, H)), wspec((D, H)), wspec((DE, H)), wspec((1, H)),
                wspec((H, 1)), wspec((1, 1)),
                wspec((H, H)), wspec((1, H)), wspec((H, 1)), wspec((1, 1))]
    out_specs = [ebs_d, ebs_c]
    return pl.pallas_call(
        _edge_mlp_body,
        grid=grid,
        in_specs=in_specs,
        out_specs=out_specs,
        out_shape=[jax.ShapeDtypeStruct((E, D), _f32),
                   jax.ShapeDtypeStruct((E, CP), _f32)],
        compiler_params=pltpu.CompilerParams(
            dimension_semantics=("arbitrary",)),
    )(src, dst, srcc, dstc, edge_attr, *w)


def _scatter_body(ef_h, tr_h, row_h, zf_h, zc_h,
                  outf_o, outc_o,
                  accf, accc, idx, bufe, bufc, tidx, tbe, tbc,
                  s1, s2):
    c = lax.axis_index("c")
    s = lax.axis_index("s")
    wid = s * NC + c
    base0 = wid * EPW

    pltpu.sync_copy(zf_h.at[pl.ds(s * RPS, RPS)], accf.at[pl.ds(s * RPS, RPS)])
    pltpu.sync_copy(zc_h.at[pl.ds(s * RPS, RPS)], accc.at[pl.ds(s * RPS, RPS)])
    plsc.subcore_barrier()

    def chunk(base, ir, be, bc, sz):
        pltpu.sync_copy(row_h.at[pl.ds(base, sz)], ir)
        d1 = pltpu.async_copy(ef_h.at[pl.ds(base, sz)], be, s1)
        d2 = pltpu.async_copy(tr_h.at[pl.ds(base, sz)], bc, s2)
        d1.wait()
        d2.wait()
        pltpu.sync_copy(be, accf.at[ir], add=True)
        pltpu.sync_copy(bc, accc.at[ir], add=True)

    def body(i, _):
        chunk(base0 + i * CH, idx, bufe, bufc, CH)
        return _

    lax.fori_loop(0, NFULL, body, None)
    if TAIL:
        chunk(base0 + NFULL * CH, tidx, tbe, tbc, TAIL)

    plsc.subcore_barrier()
    pltpu.sync_copy(accf.at[pl.ds(s * RPS, RPS)],
                    outf_o.at[c].at[pl.ds(s * RPS, RPS)])
    pltpu.sync_copy(accc.at[pl.ds(s * RPS, RPS)],
                    outc_o.at[c].at[pl.ds(s * RPS, RPS)])


def _sc_scatter(ef, tr, row, zf, zc):
    fn = pl.kernel(
        _scatter_body,
        out_type=[
            jax.ShapeDtypeStruct((NC, N, D), _f32),
            jax.ShapeDtypeStruct((NC, N, CP), _f32),
        ],
        mesh=plsc.VectorSubcoreMesh(core_axis_name="c", subcore_axis_name="s",
                                    num_cores=NC, num_subcores=NS),
        scratch_types=[
            pltpu.VMEM_SHARED((N, D), _f32),
            pltpu.VMEM_SHARED((N, CP), _f32),
            pltpu.VMEM((CH,), _i32),
            pltpu.VMEM((CH, D), _f32), pltpu.VMEM((CH, CP), _f32),
            pltpu.VMEM((TAIL,), _i32),
            pltpu.VMEM((TAIL, D), _f32), pltpu.VMEM((TAIL, CP), _f32),
            pltpu.SemaphoreType.DMA, pltpu.SemaphoreType.DMA,
        ],
        compiler_params=pltpu.CompilerParams(use_tc_tiling_on_sc=False),
    )
    return fn(ef, tr, row, zf, zc)


def _node_body(nodes_r, coordp_r, aggf_r, aggc_r,
               nW1a_r, nW1b_r, nb1_r, nW2_r, nb2_r,
               nodes_o, coordp_o):
    nodes = nodes_r[...]
    aggf = aggf_r[0] + aggf_r[1]

    def mm(a, b):
        return jnp.dot(a, b, preferred_element_type=_f32)

    n = mm(nodes, nW1a_r[...]) + mm(aggf, nW1b_r[...]) + nb1_r[...]
    n = n - jnp.tanh(n)
    n = mm(n, nW2_r[...]) + nb2_r[...]
    nodes_o[...] = nodes + n
    coordp_o[...] = coordp_r[...] + aggc_r[0] + aggc_r[1]


def _node_mlp(nodes, coordp, aggf, aggc, nW1a, nW1b, nb1, nW2, nb2):
    BN = 2000
    grid = (N // BN,)
    return pl.pallas_call(
        _node_body,
        grid=grid,
        in_specs=[
            pl.BlockSpec((BN, D), lambda i: (i, 0)),
            pl.BlockSpec((BN, CP), lambda i: (i, 0)),
            pl.BlockSpec((NC, BN, D), lambda i: (0, i, 0)),
            pl.BlockSpec((NC, BN, CP), lambda i: (0, i, 0)),
            pl.BlockSpec((D, H), lambda i: (0, 0)),
            pl.BlockSpec((H, H), lambda i: (0, 0)),
            pl.BlockSpec((1, H), lambda i: (0, 0)),
            pl.BlockSpec((H, D), lambda i: (0, 0)),
            pl.BlockSpec((1, D), lambda i: (0, 0)),
        ],
        out_specs=[
            pl.BlockSpec((BN, D), lambda i: (i, 0)),
            pl.BlockSpec((BN, CP), lambda i: (i, 0)),
        ],
        out_shape=[jax.ShapeDtypeStruct((N, D), _f32),
                   jax.ShapeDtypeStruct((N, CP), _f32)],
        compiler_params=pltpu.CompilerParams(
            dimension_semantics=("arbitrary",)),
    )(nodes, coordp, aggf, aggc, nW1a, nW1b, nb1, nW2, nb2)


def kernel(nodes, coord, edges, edge_attr,
           eW1, eb1, eW2, eb2, aW1, ab1, aW2, ab2,
           cW1, cb1, cW2, cb2, nW1, nb1, nW2, nb2):
    row = edges[0]
    col = edges[1]
    coordp = jnp.pad(coord, ((0, 0), (0, CP - 3)))

    src, dst, srcc, dstc = _sc_gather(nodes, coordp, row, col)

    w = (eW1[:D], eW1[D:2 * D], eW1[2 * D:2 * D + 1], eW1[2 * D + 1:],
         eb1[None, :], eW2, eb2[None, :],
         aW1[:D], aW1[D:2 * D], aW1[2 * D:], ab1[None, :],
         aW2, ab2[None, :],
         cW1, cb1[None, :], cW2, cb2[None, :])
    ef, tr = _edge_mlp(src, dst, srcc, dstc, edge_attr, w)

    zf = jnp.zeros((N, D), _f32)
    zc = jnp.zeros((N, CP), _f32)
    aggf, aggc = _sc_scatter(ef, tr, row, zf, zc)

    nodes_out, coordp_out = _node_mlp(nodes, coordp, aggf, aggc,
                                      nW1[:D], nW1[D:], nb1[None, :],
                                      nW2, nb2[None, :])
    return (nodes_out, coordp_out[:, :3])
